# baseline (speedup 1.0000x reference)
import jax
import jax.numpy as jnp
from jax import lax
from jax.experimental import pallas as pl
from jax.experimental.pallas import tpu as pltpu
from jax.experimental.pallas import tpu_sc as plsc

_NT = 60
_PS = 3375
_PP = 3376
_B = 16384
_NC = 2
_NS = 16
_BPT = _B // (_NC * _NS)
_KH = _NT // 2
_LANES = 16


def _sc_body(x_hbm, W_hbm, out_hbm, x_v, idx_v, vals_v, acc_v, tab_v, semx,
             semg0, semg1, tabs_sh):
    c = lax.axis_index("c")
    s = lax.axis_index("s")
    wid = c * _NS + s
    base = wid * _BPT

    x_dma = pltpu.async_copy(x_hbm.at[pl.ds(base, _BPT), :], x_v, semx)

    @pl.when(s < 8)
    def _():
        r0 = jnp.minimum(s * 8, _NT - 8)
        for h in range(2):
            pltpu.sync_copy(W_hbm.at[pl.ds(r0 + 4 * h, 4), :], tab_v)
            for j in range(4):
                pltpu.sync_copy(
                    tab_v.at[j],
                    tabs_sh.at[pl.ds((r0 + 4 * h + j) * _PP, _PP)])

    x_dma.wait()
    plsc.subcore_barrier()

    iota = lax.iota(jnp.int32, _LANES)

    def build(k0, k1):
        def bv(v, _):
            pos = pl.multiple_of(v * _LANES, _LANES)
            rows = pos + iota
            for k in range(k0, k1):
                kvec = jnp.full((_LANES,), k, jnp.int32)
                xv = plsc.load_gather(x_v, [rows, kvec])
                idx_v[pl.ds(k * _BPT + pos, _LANES)] = xv + (k * _PP)
            return 0
        lax.fori_loop(0, _BPT // _LANES, bv, 0)

    build(0, _KH)
    g0 = pltpu.async_copy(tabs_sh.at[idx_v.at[pl.ds(0, _KH * _BPT)]],
                          vals_v.at[pl.ds(0, _KH * _BPT)], semg0)
    build(_KH, _NT)
    g1 = pltpu.async_copy(
        tabs_sh.at[idx_v.at[pl.ds(_KH * _BPT, _KH * _BPT)]],
        vals_v.at[pl.ds(_KH * _BPT, _KH * _BPT)], semg1)

    def accum(k0, k1, init):
        def av(v, _):
            pos = pl.multiple_of(v * _LANES, _LANES)
            acc = acc_v[pl.ds(pos, _LANES)] if not init else jnp.zeros(
                (_LANES,), jnp.float32)
            for k in range(k0, k1):
                acc = acc + vals_v[pl.ds(k * _BPT + pos, _LANES)]
            acc_v[pl.ds(pos, _LANES)] = acc
            return 0
        lax.fori_loop(0, _BPT // _LANES, av, 0)

    g0.wait()
    accum(0, _KH, True)
    g1.wait()
    accum(_KH, _NT, False)

    pltpu.sync_copy(acc_v, out_hbm.at[pl.ds(base, _BPT)])


@jax.jit
def _sc_call(x, W2):
    mesh = plsc.VectorSubcoreMesh(
        core_axis_name="c", subcore_axis_name="s",
        num_cores=_NC, num_subcores=_NS)
    f = pl.kernel(
        _sc_body,
        out_type=jax.ShapeDtypeStruct((_B,), jnp.float32),
        mesh=mesh,
        scratch_types=[
            pltpu.VMEM((_BPT, _NT), jnp.int32),
            pltpu.VMEM((_NT * _BPT,), jnp.int32),
            pltpu.VMEM((_NT * _BPT,), jnp.float32),
            pltpu.VMEM((_BPT,), jnp.float32),
            pltpu.VMEM((4, _PP), jnp.float32),
            pltpu.SemaphoreType.DMA,
            pltpu.SemaphoreType.DMA,
            pltpu.SemaphoreType.DMA,
            pltpu.VMEM_SHARED((_NT * _PP,), jnp.float32),
        ],
        compiler_params=pltpu.CompilerParams(
            use_tc_tiling_on_sc=False, needs_layout_passes=False),
    )
    return f(x, W2)


def kernel(x, W):
    Wp = jnp.pad(W[:, :, 0], ((0, 0), (0, _PP - _PS)))
    y = _sc_call(x.astype(jnp.int32), Wp)
    return y[:, None]


# speedup vs baseline: 2.3105x; 2.3105x over previous
import jax
import jax.numpy as jnp
from jax import lax
from jax.experimental import pallas as pl
from jax.experimental.pallas import tpu as pltpu
from jax.experimental.pallas import tpu_sc as plsc

_NT = 60
_PS = 3375
_B = 16384
_NC = 2
_NS = 16
_TG = 8
_BG = 4
_TPG = 8
_BPG = _B // _BG
_HALF = _BPG // 2
_STRIPE = _BPG // _TG
_LANES = 16


def _sc_body(x_hbm, W_hbm, out_hbm, tab_v, x_v, acc_v, tmp_v, sem0, sem1,
             sem2, shared):
    c = lax.axis_index("c")
    s = lax.axis_index("s")
    tg = s % _TG
    bg = c * (_NS // _TG) + s // _TG
    sbase = s - tg
    off = jnp.where(tg == _TG - 1, _NT - _TPG, tg * _TPG)
    nown = jnp.where(tg == _TG - 2, _NT - (_TG - 1) * _TPG, _TPG)

    roa, ria = off // 8, off % 8
    rob, rib = (off + 4) // 8, (off + 4) % 8
    co0 = bg * (_BPG // 128)

    tab_dma = pltpu.async_copy(W_hbm.at[pl.ds(off, _TPG), :], tab_v, sem0)
    x_dmas = []
    for h, sem in ((0, sem1), (1, sem2)):
        x_dmas.append(pltpu.async_copy(
            x_hbm.at[roa, pl.ds(co0 + 16 * h, 16), pl.ds(ria, 4), :],
            x_v.at[pl.ds(16 * h, 16), pl.ds(0, 4), :], sem))
        x_dmas.append(pltpu.async_copy(
            x_hbm.at[rob, pl.ds(co0 + 16 * h, 16), pl.ds(rib, 4), :],
            x_v.at[pl.ds(16 * h, 16), pl.ds(4, 4), :], sem))

    def body(v, _):
        co = v // 8
        ci0 = pl.multiple_of((v % 8) * _LANES, _LANES)
        acc = jnp.zeros((_LANES,), jnp.float32)
        for k in range(_TPG):
            kvec = jnp.full((_LANES,), k, jnp.int32)
            xv = x_v[co, k, pl.ds(ci0, _LANES)]
            val = plsc.load_gather(tab_v, [kvec, xv])
            acc = acc + jnp.where(k < nown, val, 0.0)
        acc_v[pl.ds(v * _LANES, _LANES)] = acc
        return 0

    tab_dma.wait()
    x_dmas[0].wait()
    x_dmas[1].wait()
    lax.fori_loop(0, _HALF // _LANES, body, 0)
    x_dmas[2].wait()
    x_dmas[3].wait()
    lax.fori_loop(_HALF // _LANES, _BPG // _LANES, body, 0)

    pltpu.sync_copy(acc_v, shared.at[s])
    plsc.subcore_barrier()

    for j in range(_TG):
        pltpu.sync_copy(shared.at[sbase + j, pl.ds(tg * _STRIPE, _STRIPE)],
                        tmp_v.at[j])

    def red(v, _):
        pos = pl.multiple_of(v * _LANES, _LANES)
        tot = jnp.zeros((_LANES,), jnp.float32)
        for j in range(_TG):
            tot = tot + tmp_v[j, pl.ds(pos, _LANES)]
        acc_v[pl.ds(pos, _LANES)] = tot
        return 0

    lax.fori_loop(0, _STRIPE // _LANES, red, 0)
    pltpu.sync_copy(acc_v.at[pl.ds(0, _STRIPE)],
                    out_hbm.at[pl.ds(bg * _BPG + tg * _STRIPE, _STRIPE)])


@jax.jit
def _sc_call(xT, W2):
    mesh = plsc.VectorSubcoreMesh(
        core_axis_name="c", subcore_axis_name="s",
        num_cores=_NC, num_subcores=_NS)
    f = pl.kernel(
        _sc_body,
        out_type=jax.ShapeDtypeStruct((_B,), jnp.float32),
        mesh=mesh,
        scratch_types=[
            pltpu.VMEM((_TPG, _PS), jnp.float32),
            pltpu.VMEM((_BPG // 128, _TPG, 128), jnp.int32),
            pltpu.VMEM((_BPG,), jnp.float32),
            pltpu.VMEM((_TG, _STRIPE), jnp.float32),
            pltpu.SemaphoreType.DMA,
            pltpu.SemaphoreType.DMA,
            pltpu.SemaphoreType.DMA,
            pltpu.VMEM_SHARED((_NS, _BPG), jnp.float32),
        ],
        compiler_params=pltpu.CompilerParams(
            use_tc_tiling_on_sc=False, needs_layout_passes=False),
    )
    return f(xT, W2)


def kernel(x, W):
    W2 = W.reshape(_NT, _PS)
    xp = jnp.pad(x.astype(jnp.int32), ((0, 0), (0, 4)))
    xt4 = xp.reshape(128, 128, 8, 8).transpose(2, 0, 3, 1)
    y = _sc_call(xt4, W2)
    return y[:, None]
